# baseline (speedup 1.0000x reference)
import jax
import jax.numpy as jnp
from jax import lax
from jax.experimental import pallas as pl
from jax.experimental.pallas import tpu as pltpu
from jax.experimental.pallas import tpu_sc as plsc

B = 16384
F = 32
NC = 2
NS = 16
NW = NC * NS
BPW = B // NW
CHUNK = 128
NCHUNK = BPW // CHUNK
GROUPS = BPW // 16
NSLOT = 8
LAG = 7


def _dot_body(uid_hbm, iid_hbm, uet_hbm, iet_hbm, ub_hbm, ib_hbm, out_hbm,
              uidv, iidv, ublk, iblk, ubw, ibw, pbuf, outv,
              usem, isem, ubsem, ibsem):
    wid = lax.axis_index("s") * NC + lax.axis_index("c")
    base = wid * BPW

    pltpu.sync_copy(uid_hbm.at[pl.ds(base, BPW)], uidv)
    pltpu.sync_copy(iid_hbm.at[pl.ds(base, BPW)], iidv)

    iota16 = lax.iota(jnp.int32, 16)

    def fire(slot, bu, bi):
        pltpu.async_copy(uet_hbm.at[:, pl.ds((bu >> 7) * 128, 128)],
                         ublk.at[slot], usem)
        pltpu.async_copy(iet_hbm.at[:, pl.ds((bi >> 7) * 128, 128)],
                         iblk.at[slot], isem)
        pltpu.async_copy(ub_hbm.at[pl.ds((bu >> 3) * 8, 8)],
                         ubw.at[slot], ubsem)
        pltpu.async_copy(ib_hbm.at[pl.ds((bi >> 3) * 8, 8)],
                         ibw.at[slot], ibsem)

    lane0 = jnp.where(iota16 == 0, jnp.float32(1.0), jnp.float32(0.0))

    def extract(slot, n, bu, bi):
        pltpu.make_async_copy(uet_hbm.at[:, pl.ds(0, 128)],
                              ublk.at[slot], usem).wait()
        pltpu.make_async_copy(iet_hbm.at[:, pl.ds(0, 128)],
                              iblk.at[slot], isem).wait()
        pltpu.make_async_copy(ub_hbm.at[pl.ds(0, 8)],
                              ubw.at[slot], ubsem).wait()
        pltpu.make_async_copy(ib_hbm.at[pl.ds(0, 8)],
                              ibw.at[slot], ibsem).wait()
        cu = jnp.zeros((16,), jnp.int32) + (bu & 127)
        ci = jnp.zeros((16,), jnp.int32) + (bi & 127)
        bwu = plsc.load_gather(ubw.at[slot], [jnp.zeros((16,), jnp.int32) + (bu & 7)])
        bwi = plsc.load_gather(ibw.at[slot], [jnp.zeros((16,), jnp.int32) + (bi & 7)])
        acc = (bwu + bwi) * lane0
        for h in range(2):
            rows = iota16 + h * 16
            uv = plsc.load_gather(ublk.at[slot], [rows, cu])
            iv = plsc.load_gather(iblk.at[slot], [rows, ci])
            acc = acc + uv * iv
        pbuf[pl.ds(n * 16, 16)] = acc

    zvec = jnp.zeros((16,), jnp.int32)

    def step(g, carry):
        uprev, iprev = carry
        ucur = uidv[pl.ds(g * 16, 16)]
        icur = iidv[pl.ds(g * 16, 16)]
        for i in range(16):
            fire(i % NSLOT, ucur[i], icur[i])
            m = i - LAG
            if m >= 0:
                extract(m % NSLOT, g * 16 + m, ucur[m], icur[m])
            else:
                mp = m + 16

                @pl.when(g > 0)
                def _():
                    extract(mp % NSLOT, (g - 1) * 16 + mp,
                            uprev[mp], iprev[mp])
        return ucur, icur

    ulast, ilast = lax.fori_loop(0, GROUPS, step, (zvec, zvec),
                                 unroll=False)
    for mp in range(16 - LAG, 16):
        extract(mp % NSLOT, (GROUPS - 1) * 16 + mp, ulast[mp], ilast[mp])

    def group(g, carry):
        acc = jnp.zeros((16,), jnp.float32)
        rows = (g * 16 + iota16) * 16
        for k in range(16):
            acc = acc + plsc.load_gather(pbuf, [rows + k])
        outv[pl.ds(g * 16, 16)] = acc
        return carry

    lax.fori_loop(0, GROUPS, group, 0)
    pltpu.sync_copy(outv, out_hbm.at[pl.ds(base, BPW)])


@jax.jit
def _mf(uid, iid, uet, iet, ub, ib):
    mesh = plsc.VectorSubcoreMesh(core_axis_name="c", subcore_axis_name="s")
    dot_run = pl.kernel(
        _dot_body,
        mesh=mesh,
        compiler_params=pltpu.CompilerParams(
            needs_layout_passes=False, use_tc_tiling_on_sc=True),
        out_type=jax.ShapeDtypeStruct((B,), jnp.float32),
        scratch_types=[
            pltpu.VMEM((BPW,), jnp.int32),
            pltpu.VMEM((BPW,), jnp.int32),
            pltpu.VMEM((NSLOT, F, 128), jnp.float32),
            pltpu.VMEM((NSLOT, F, 128), jnp.float32),
            pltpu.VMEM((NSLOT, 8), jnp.float32),
            pltpu.VMEM((NSLOT, 8), jnp.float32),
            pltpu.VMEM((BPW * 16,), jnp.float32),
            pltpu.VMEM((BPW,), jnp.float32),
            pltpu.SemaphoreType.DMA,
            pltpu.SemaphoreType.DMA,
            pltpu.SemaphoreType.DMA,
            pltpu.SemaphoreType.DMA,
        ],
    )
    return dot_run(uid, iid, uet, iet, ub, ib)


def kernel(user_ids, item_ids, user_embed, user_bias_embed, item_embed,
           item_bias_embed):
    uid = user_ids.astype(jnp.int32)
    iid = item_ids.astype(jnp.int32)
    uet = user_embed.T
    iet = item_embed.T
    ub = user_bias_embed.reshape(-1)
    ib = item_bias_embed.reshape(-1)
    out = _mf(uid, iid, uet, iet, ub, ib)
    return out.reshape(B, 1)


# speedup vs baseline: 1.3526x; 1.3526x over previous
import jax
import jax.numpy as jnp
from jax import lax
from jax.experimental import pallas as pl
from jax.experimental.pallas import tpu as pltpu
from jax.experimental.pallas import tpu_sc as plsc

B = 16384
F = 32
NC = 2
NS = 16
NW = NC * NS
BPW = B // NW
CHUNK = 128
NCHUNK = BPW // CHUNK
GROUPS = BPW // 16
NSLOT = 8
LAG = 7


def _dot_body(uid_hbm, iid_hbm, uet_hbm, iet_hbm, out_hbm,
              uidv, iidv, ublk, iblk, pbuf, outv, usem, isem):
    wid = lax.axis_index("s") * NC + lax.axis_index("c")
    base = wid * BPW

    pltpu.sync_copy(uid_hbm.at[pl.ds(base, BPW)], uidv)
    pltpu.sync_copy(iid_hbm.at[pl.ds(base, BPW)], iidv)

    iota16 = lax.iota(jnp.int32, 16)

    def fire(slot, bu, bi):
        pltpu.async_copy(uet_hbm.at[:, pl.ds((bu >> 7) * 128, 128)],
                         ublk.at[slot], usem)
        pltpu.async_copy(iet_hbm.at[:, pl.ds((bi >> 7) * 128, 128)],
                         iblk.at[slot], isem)

    def extract(slot, n, bu, bi):
        pltpu.make_async_copy(uet_hbm.at[:, pl.ds(0, 128)],
                              ublk.at[slot], usem).wait()
        pltpu.make_async_copy(iet_hbm.at[:, pl.ds(0, 128)],
                              iblk.at[slot], isem).wait()
        cu = jnp.zeros((16,), jnp.int32) + (bu & 127)
        ci = jnp.zeros((16,), jnp.int32) + (bi & 127)
        acc = jnp.zeros((16,), jnp.float32)
        for h in range(2):
            rows = iota16 + h * 16
            uv = plsc.load_gather(ublk.at[slot], [rows, cu])
            iv = plsc.load_gather(iblk.at[slot], [rows, ci])
            acc = acc + uv * iv
        pbuf[pl.ds(n * 16, 16)] = acc

    zvec = jnp.zeros((16,), jnp.int32)

    def step(g, carry):
        uprev, iprev = carry
        ucur = uidv[pl.ds(g * 16, 16)]
        icur = iidv[pl.ds(g * 16, 16)]
        for i in range(16):
            fire(i % NSLOT, ucur[i], icur[i])
            m = i - LAG
            if m >= 0:
                extract(m % NSLOT, g * 16 + m, ucur[m], icur[m])
            else:
                mp = m + 16

                @pl.when(g > 0)
                def _():
                    extract(mp % NSLOT, (g - 1) * 16 + mp,
                            uprev[mp], iprev[mp])
        return ucur, icur

    ulast, ilast = lax.fori_loop(0, GROUPS, step, (zvec, zvec),
                                 unroll=False)
    for mp in range(16 - LAG, 16):
        extract(mp % NSLOT, (GROUPS - 1) * 16 + mp, ulast[mp], ilast[mp])

    def group(g, carry):
        acc = jnp.zeros((16,), jnp.float32)
        rows = (g * 16 + iota16) * 16
        for k in range(16):
            acc = acc + plsc.load_gather(pbuf, [rows + k])
        outv[pl.ds(g * 16, 16)] = acc
        return carry

    lax.fori_loop(0, GROUPS, group, 0)
    pltpu.sync_copy(outv, out_hbm.at[pl.ds(base, BPW)])


def _bias_body(dot_hbm, uid_hbm, iid_hbm, ub_hbm, ib_hbm, out_hbm,
               uidx, iidx, dotv, ubv, ibv, outv, sem):
    wid = lax.axis_index("s") * NC + lax.axis_index("c")
    base = wid * BPW

    pltpu.sync_copy(uid_hbm.at[wid], uidx)
    pltpu.sync_copy(iid_hbm.at[wid], iidx)
    pltpu.sync_copy(dot_hbm.at[pl.ds(base, BPW)], dotv)

    copies = []
    for j in range(NCHUNK):
        sl = pl.ds(j * CHUNK, CHUNK)
        copies.append(pltpu.async_copy(ub_hbm.at[uidx.at[j]], ubv.at[sl], sem))
        copies.append(pltpu.async_copy(ib_hbm.at[iidx.at[j]], ibv.at[sl], sem))
    for cp in copies:
        cp.wait()

    def group(g, carry):
        sl = pl.ds(g * 16, 16)
        outv[sl] = dotv[sl] + ubv[sl] + ibv[sl]
        return carry

    lax.fori_loop(0, GROUPS, group, 0)
    pltpu.sync_copy(outv, out_hbm.at[pl.ds(base, BPW)])


@jax.jit
def _mf(uid, iid, uet, iet, ub, ib):
    mesh = plsc.VectorSubcoreMesh(core_axis_name="c", subcore_axis_name="s")
    dot_run = pl.kernel(
        _dot_body,
        mesh=mesh,
        compiler_params=pltpu.CompilerParams(
            needs_layout_passes=False, use_tc_tiling_on_sc=True),
        out_type=jax.ShapeDtypeStruct((B,), jnp.float32),
        scratch_types=[
            pltpu.VMEM((BPW,), jnp.int32),
            pltpu.VMEM((BPW,), jnp.int32),
            pltpu.VMEM((NSLOT, F, 128), jnp.float32),
            pltpu.VMEM((NSLOT, F, 128), jnp.float32),
            pltpu.VMEM((BPW * 16,), jnp.float32),
            pltpu.VMEM((BPW,), jnp.float32),
            pltpu.SemaphoreType.DMA,
            pltpu.SemaphoreType.DMA,
        ],
    )
    bias_run = pl.kernel(
        _bias_body,
        mesh=mesh,
        compiler_params=pltpu.CompilerParams(
            needs_layout_passes=False, use_tc_tiling_on_sc=False),
        out_type=jax.ShapeDtypeStruct((B,), jnp.float32),
        scratch_types=[
            pltpu.VMEM((NCHUNK, CHUNK), jnp.int32),
            pltpu.VMEM((NCHUNK, CHUNK), jnp.int32),
            pltpu.VMEM((BPW,), jnp.float32),
            pltpu.VMEM((BPW,), jnp.float32),
            pltpu.VMEM((BPW,), jnp.float32),
            pltpu.VMEM((BPW,), jnp.float32),
            pltpu.SemaphoreType.DMA,
        ],
    )
    dots = dot_run(uid, iid, uet, iet)
    return bias_run(dots, uid.reshape(NW, NCHUNK, CHUNK),
                    iid.reshape(NW, NCHUNK, CHUNK), ub, ib)


def kernel(user_ids, item_ids, user_embed, user_bias_embed, item_embed,
           item_bias_embed):
    uid = user_ids.astype(jnp.int32)
    iid = item_ids.astype(jnp.int32)
    uet = user_embed.T
    iet = item_embed.T
    ub = user_bias_embed.reshape(-1)
    ib = item_bias_embed.reshape(-1)
    out = _mf(uid, iid, uet, iet, ub, ib)
    return out.reshape(B, 1)
